# Initial kernel scaffold; baseline (speedup 1.0000x reference)
import jax
import jax.numpy as jnp
from jax.experimental import pallas as pl


def kernel(x, Wg, bg, W_gate, b_gate, W_up, b_up, W_down, b_down):
    raise NotImplementedError("write your pallas kernel here")


# speedup vs baseline: 1.3678x; 1.3678x over previous
import functools

import jax
import jax.numpy as jnp
from jax import lax
from jax.experimental import pallas as pl
from jax.experimental.pallas import tpu as pltpu

DM = 1024
DF = 2048
NE = 8
TOPK = 2
T = 8192
A = T * TOPK
BLK = 512
NG = A // BLK + NE
S = NG * BLK
TB = 512


def _router_body(x_ref, wg_ref, bg_ref, w0_ref, w1_ref, i0_ref, i1_ref):
    logits = jnp.dot(x_ref[...], wg_ref[...],
                     preferred_element_type=jnp.float32) + bg_ref[0]
    cols = lax.broadcasted_iota(jnp.int32, (TB, NE), 1)
    i0 = jnp.argmax(logits, axis=1).astype(jnp.int32)
    m0 = jnp.max(logits, axis=1)
    masked = jnp.where(cols == i0[:, None], -jnp.inf, logits)
    i1 = jnp.argmax(masked, axis=1).astype(jnp.int32)
    m1 = jnp.max(masked, axis=1)
    w0 = jax.nn.sigmoid(m0 - m1)
    w0_ref[...] = w0
    w1_ref[...] = 1.0 - w0
    i0_ref[...] = i0
    i1_ref[...] = i1


def _router(xf, Wg, bg):
    grid = (T // TB,)
    out = pl.pallas_call(
        _router_body,
        grid=grid,
        in_specs=[
            pl.BlockSpec((TB, DM), lambda i: (i, 0)),
            pl.BlockSpec((DM, NE), lambda i: (0, 0)),
            pl.BlockSpec((1, NE), lambda i: (0, 0)),
        ],
        out_specs=[
            pl.BlockSpec((TB,), lambda i: (i,)),
            pl.BlockSpec((TB,), lambda i: (i,)),
            pl.BlockSpec((TB,), lambda i: (i,)),
            pl.BlockSpec((TB,), lambda i: (i,)),
        ],
        out_shape=[
            jax.ShapeDtypeStruct((T,), jnp.float32),
            jax.ShapeDtypeStruct((T,), jnp.float32),
            jax.ShapeDtypeStruct((T,), jnp.int32),
            jax.ShapeDtypeStruct((T,), jnp.int32),
        ],
    )(xf, Wg, bg.reshape(1, NE))
    return out


def _ffn_body(be_ref, xg_ref, wg_ref, bg_ref, wu_ref, bu_ref, wd_ref, bd_ref,
              ws_ref, og_ref):
    xb = xg_ref[...]
    g = jnp.dot(xb, wg_ref[0], preferred_element_type=jnp.float32) + bg_ref[0]
    u = jnp.dot(xb, wu_ref[0], preferred_element_type=jnp.float32) + bu_ref[0]
    t = g * u
    h = (t * jax.nn.sigmoid(t)).astype(jnp.bfloat16)
    o = jnp.dot(h, wd_ref[0], preferred_element_type=jnp.float32) + bd_ref[0]
    og_ref[...] = o * ws_ref[...]


def _ffn(xg, W_gate, b_gate, W_up, b_up, W_down, b_down, w_slot, block_expert):
    grid_spec = pltpu.PrefetchScalarGridSpec(
        num_scalar_prefetch=1,
        grid=(NG,),
        in_specs=[
            pl.BlockSpec((BLK, DM), lambda g, be: (g, 0)),
            pl.BlockSpec((1, DM, DF), lambda g, be: (be[g], 0, 0)),
            pl.BlockSpec((1, 1, DF), lambda g, be: (be[g], 0, 0)),
            pl.BlockSpec((1, DM, DF), lambda g, be: (be[g], 0, 0)),
            pl.BlockSpec((1, 1, DF), lambda g, be: (be[g], 0, 0)),
            pl.BlockSpec((1, DF, DM), lambda g, be: (be[g], 0, 0)),
            pl.BlockSpec((1, 1, DM), lambda g, be: (be[g], 0, 0)),
            pl.BlockSpec((BLK, 1), lambda g, be: (g, 0)),
        ],
        out_specs=pl.BlockSpec((BLK, DM), lambda g, be: (g, 0)),
    )
    return pl.pallas_call(
        _ffn_body,
        grid_spec=grid_spec,
        out_shape=jax.ShapeDtypeStruct((S, DM), jnp.float32),
    )(block_expert, xg, W_gate, b_gate.reshape(NE, 1, DF), W_up,
      b_up.reshape(NE, 1, DF), W_down, b_down.reshape(NE, 1, DM),
      w_slot.reshape(S, 1))


def _dispatch(i0, i1, w0, w1):
    e_flat = jnp.stack([i0, i1], axis=1).reshape(-1)
    w_flat = jnp.stack([w0, w1], axis=1).reshape(-1)
    onehot = (e_flat[:, None] == jnp.arange(NE, dtype=jnp.int32)[None, :]
              ).astype(jnp.int32)
    inc = jnp.cumsum(onehot, axis=0)
    rank = jnp.sum((inc - onehot) * onehot, axis=1)
    counts = inc[-1]
    bpe = (counts + BLK - 1) // BLK
    block_first = jnp.concatenate(
        [jnp.zeros((1,), jnp.int32), jnp.cumsum(bpe)[:-1]]).astype(jnp.int32)
    block_of_a = block_first[e_flat] + rank // BLK
    slot_of_a = block_of_a * BLK + rank % BLK
    tok_of_a = jnp.arange(A, dtype=jnp.int32) // TOPK
    tok_slot = jnp.zeros((S,), jnp.int32).at[slot_of_a].set(tok_of_a)
    w_slot = jnp.zeros((S,), jnp.float32).at[slot_of_a].set(w_flat)
    block_expert = jnp.full((NG,), NE - 1, jnp.int32).at[block_of_a].set(e_flat)
    pos0 = slot_of_a[0::2]
    pos1 = slot_of_a[1::2]
    return tok_slot, w_slot, block_expert, pos0, pos1


def kernel(x, Wg, bg, W_gate, b_gate, W_up, b_up, W_down, b_down):
    bs, slen, dim = x.shape
    xf = x.reshape(-1, dim)
    w0, w1, i0, i1 = _router(xf, Wg, bg)
    tok_slot, w_slot, block_expert, pos0, pos1 = _dispatch(i0, i1, w0, w1)
    xbf = xf.astype(jnp.bfloat16)
    xg = xbf[tok_slot]
    og = _ffn(xg, W_gate.astype(jnp.bfloat16), b_gate,
              W_up.astype(jnp.bfloat16), b_up,
              W_down.astype(jnp.bfloat16), b_down, w_slot, block_expert)
    y = og[pos0] + og[pos1]
    return y.reshape(bs, slen, dim)


# speedup vs baseline: 1.4752x; 1.0785x over previous
import functools

import jax
import jax.numpy as jnp
from jax import lax
from jax.experimental import pallas as pl
from jax.experimental.pallas import tpu as pltpu

DM = 1024
DF = 2048
NE = 8
TOPK = 2
T = 8192
A = T * TOPK
BLK = 512
NG = A // BLK + NE
S = NG * BLK
TB = 512


NTB = T // TB


def _route_body(x_ref, wg_ref, bg_ref,
                w0_ref, w1_ref, p0_ref, p1_ref, be_ref,
                e0s, e1s, w0s, w1s, cnt, cnt2):
    p = pl.program_id(0)
    g = pl.program_id(1)
    cols = lax.broadcasted_iota(jnp.int32, (TB, NE), 1)

    @pl.when(p == 0)
    def _phase0():
        logits = jnp.dot(x_ref[...], wg_ref[...],
                         preferred_element_type=jnp.float32) + bg_ref[0]
        i0 = jnp.argmax(logits, axis=1).astype(jnp.int32)
        m0 = jnp.max(logits, axis=1)
        masked = jnp.where(cols == i0[:, None], -jnp.inf, logits)
        i1 = jnp.argmax(masked, axis=1).astype(jnp.int32)
        m1 = jnp.max(masked, axis=1)
        w0 = jax.nn.sigmoid(m0 - m1)
        e0s[g, :] = i0
        e1s[g, :] = i1
        w0s[g, :] = w0
        w1s[g, :] = 1.0 - w0
        oh = ((cols == i0[:, None]).astype(jnp.float32)
              + (cols == i1[:, None]).astype(jnp.float32))
        colsum = jnp.sum(oh, axis=0, keepdims=True)
        prev = jnp.where(g == 0, jnp.zeros_like(cnt[...]), cnt[...])
        cnt[...] = prev + colsum

    @pl.when(p == 1)
    def _phase1():
        tot_i = cnt[...].astype(jnp.int32)
        bpe = (tot_i + BLK - 1) >> 9
        iu = lax.broadcasted_iota(jnp.int32, (NE, NE), 0)
        ju = lax.broadcasted_iota(jnp.int32, (NE, NE), 1)
        um = (iu < ju).astype(jnp.float32)
        bf = jnp.dot(bpe.astype(jnp.float32), um,
                     preferred_element_type=jnp.float32)
        prev2 = jnp.where(g == 0, jnp.zeros_like(cnt2[...]), cnt2[...])
        i0 = e0s[g, :]
        i1 = e1s[g, :]
        oh0 = (cols == i0[:, None]).astype(jnp.float32)
        oh1 = (cols == i1[:, None]).astype(jnp.float32)
        ohs = oh0 + oh1
        ri = lax.broadcasted_iota(jnp.int32, (TB, TB), 0)
        ci = lax.broadcasted_iota(jnp.int32, (TB, TB), 1)
        stril = (ci < ri).astype(jnp.bfloat16)
        cumbt = jnp.dot(stril, ohs.astype(jnp.bfloat16),
                        preferred_element_type=jnp.float32)
        base = prev2 + cumbt
        r0 = jnp.sum(base * oh0, axis=1).astype(jnp.int32)
        r1 = jnp.sum(base * oh1, axis=1).astype(jnp.int32)
        bfb = jnp.broadcast_to(bf, (TB, NE))
        bf0 = jnp.sum(bfb * oh0, axis=1).astype(jnp.int32)
        bf1 = jnp.sum(bfb * oh1, axis=1).astype(jnp.int32)
        p0_ref[...] = ((bf0 + (r0 >> 9)) << 9) + (r0 & (BLK - 1))
        p1_ref[...] = ((bf1 + (r1 >> 9)) << 9) + (r1 & (BLK - 1))
        w0_ref[...] = w0s[g, :]
        w1_ref[...] = w1s[g, :]
        cnt2[...] = prev2 + jnp.sum(ohs, axis=0, keepdims=True)
        bidx = lax.broadcasted_iota(jnp.int32, (NG, NE), 0)
        bfg = jnp.broadcast_to(bf, (NG, NE)).astype(jnp.int32)
        be_ref[...] = jnp.clip(
            jnp.sum((bidx >= bfg).astype(jnp.int32), axis=1) - 1, 0, NE - 1)


def _route(xf, Wg, bg):
    return pl.pallas_call(
        _route_body,
        grid=(2, NTB),
        in_specs=[
            pl.BlockSpec((TB, DM), lambda p, g: (g * (1 - p), 0)),
            pl.BlockSpec((DM, NE), lambda p, g: (0, 0)),
            pl.BlockSpec((1, NE), lambda p, g: (0, 0)),
        ],
        out_specs=[
            pl.BlockSpec((TB,), lambda p, g: (g,)),
            pl.BlockSpec((TB,), lambda p, g: (g,)),
            pl.BlockSpec((TB,), lambda p, g: (g,)),
            pl.BlockSpec((TB,), lambda p, g: (g,)),
            pl.BlockSpec((NG,), lambda p, g: (0,)),
        ],
        out_shape=[
            jax.ShapeDtypeStruct((T,), jnp.float32),
            jax.ShapeDtypeStruct((T,), jnp.float32),
            jax.ShapeDtypeStruct((T,), jnp.int32),
            jax.ShapeDtypeStruct((T,), jnp.int32),
            jax.ShapeDtypeStruct((NG,), jnp.int32),
        ],
        scratch_shapes=[
            pltpu.VMEM((NTB, TB), jnp.int32),
            pltpu.VMEM((NTB, TB), jnp.int32),
            pltpu.VMEM((NTB, TB), jnp.float32),
            pltpu.VMEM((NTB, TB), jnp.float32),
            pltpu.VMEM((1, NE), jnp.float32),
            pltpu.VMEM((1, NE), jnp.float32),
        ],
    )(xf, Wg, bg.reshape(1, NE))


def _ffn_body(be_ref, xg_ref, wg_ref, bg_ref, wu_ref, bu_ref, wd_ref, bd_ref,
              ws_ref, og_ref):
    xb = xg_ref[...]
    g = jnp.dot(xb, wg_ref[0], preferred_element_type=jnp.float32) + bg_ref[0]
    u = jnp.dot(xb, wu_ref[0], preferred_element_type=jnp.float32) + bu_ref[0]
    t = g * u
    h = (t * jax.nn.sigmoid(t)).astype(jnp.bfloat16)
    o = jnp.dot(h, wd_ref[0], preferred_element_type=jnp.float32) + bd_ref[0]
    og_ref[...] = o * ws_ref[...]


def _ffn(xg, W_gate, b_gate, W_up, b_up, W_down, b_down, w_slot, block_expert):
    grid_spec = pltpu.PrefetchScalarGridSpec(
        num_scalar_prefetch=1,
        grid=(NG,),
        in_specs=[
            pl.BlockSpec((BLK, DM), lambda g, be: (g, 0)),
            pl.BlockSpec((1, DM, DF), lambda g, be: (be[g], 0, 0)),
            pl.BlockSpec((1, 1, DF), lambda g, be: (be[g], 0, 0)),
            pl.BlockSpec((1, DM, DF), lambda g, be: (be[g], 0, 0)),
            pl.BlockSpec((1, 1, DF), lambda g, be: (be[g], 0, 0)),
            pl.BlockSpec((1, DF, DM), lambda g, be: (be[g], 0, 0)),
            pl.BlockSpec((1, 1, DM), lambda g, be: (be[g], 0, 0)),
            pl.BlockSpec((BLK, 1), lambda g, be: (g, 0)),
        ],
        out_specs=pl.BlockSpec((BLK, DM), lambda g, be: (g, 0)),
    )
    return pl.pallas_call(
        _ffn_body,
        grid_spec=grid_spec,
        out_shape=jax.ShapeDtypeStruct((S, DM), jnp.float32),
    )(block_expert, xg, W_gate, b_gate.reshape(NE, 1, DF), W_up,
      b_up.reshape(NE, 1, DF), W_down, b_down.reshape(NE, 1, DM),
      w_slot.reshape(S, 1))


def kernel(x, Wg, bg, W_gate, b_gate, W_up, b_up, W_down, b_down):
    bs, slen, dim = x.shape
    xf = x.reshape(-1, dim)
    w0, w1, pos0, pos1, block_expert = _route(xf, Wg, bg)
    tok = jnp.arange(T, dtype=jnp.int32)
    tok_slot = (jnp.zeros((S,), jnp.int32).at[pos0].set(tok)
                .at[pos1].set(tok))
    w_slot = (jnp.zeros((S,), jnp.float32).at[pos0].set(w0)
              .at[pos1].set(w1))
    xbf = xf.astype(jnp.bfloat16)
    xg = xbf[tok_slot]
    og = _ffn(xg, W_gate.astype(jnp.bfloat16), b_gate,
              W_up.astype(jnp.bfloat16), b_up,
              W_down.astype(jnp.bfloat16), b_down, w_slot, block_expert)
    y = og[pos0] + og[pos1]
    return y.reshape(bs, slen, dim)


# speedup vs baseline: 1.9435x; 1.3175x over previous
import functools

import jax
import jax.numpy as jnp
from jax import lax
from jax.experimental import pallas as pl
from jax.experimental.pallas import tpu as pltpu
from jax.experimental.pallas import tpu_sc as plsc

DM = 1024
DF = 2048
NE = 8
TOPK = 2
T = 8192
A = T * TOPK
BLK = 512
NG = A // BLK + NE
S = NG * BLK
TB = 512


NTB = T // TB


def _route_body(x_ref, wg_ref, bg_ref,
                w0_ref, w1_ref, p0_ref, p1_ref, be_ref,
                e0s, e1s, w0s, w1s, cnt, cnt2):
    p = pl.program_id(0)
    g = pl.program_id(1)
    cols = lax.broadcasted_iota(jnp.int32, (TB, NE), 1)

    @pl.when(p == 0)
    def _phase0():
        logits = jnp.dot(x_ref[...], wg_ref[...],
                         preferred_element_type=jnp.float32) + bg_ref[0]
        i0 = jnp.argmax(logits, axis=1).astype(jnp.int32)
        m0 = jnp.max(logits, axis=1)
        masked = jnp.where(cols == i0[:, None], -jnp.inf, logits)
        i1 = jnp.argmax(masked, axis=1).astype(jnp.int32)
        m1 = jnp.max(masked, axis=1)
        w0 = jax.nn.sigmoid(m0 - m1)
        e0s[g, :] = i0
        e1s[g, :] = i1
        w0s[g, :] = w0
        w1s[g, :] = 1.0 - w0
        oh = ((cols == i0[:, None]).astype(jnp.float32)
              + (cols == i1[:, None]).astype(jnp.float32))
        colsum = jnp.sum(oh, axis=0, keepdims=True)
        prev = jnp.where(g == 0, jnp.zeros_like(cnt[...]), cnt[...])
        cnt[...] = prev + colsum

    @pl.when(p == 1)
    def _phase1():
        tot_i = cnt[...].astype(jnp.int32)
        bpe = (tot_i + BLK - 1) >> 9
        iu = lax.broadcasted_iota(jnp.int32, (NE, NE), 0)
        ju = lax.broadcasted_iota(jnp.int32, (NE, NE), 1)
        um = (iu < ju).astype(jnp.float32)
        bf = jnp.dot(bpe.astype(jnp.float32), um,
                     preferred_element_type=jnp.float32)
        prev2 = jnp.where(g == 0, jnp.zeros_like(cnt2[...]), cnt2[...])
        i0 = e0s[g, :]
        i1 = e1s[g, :]
        oh0 = (cols == i0[:, None]).astype(jnp.float32)
        oh1 = (cols == i1[:, None]).astype(jnp.float32)
        ohs = oh0 + oh1
        ri = lax.broadcasted_iota(jnp.int32, (TB, TB), 0)
        ci = lax.broadcasted_iota(jnp.int32, (TB, TB), 1)
        stril = (ci < ri).astype(jnp.bfloat16)
        cumbt = jnp.dot(stril, ohs.astype(jnp.bfloat16),
                        preferred_element_type=jnp.float32)
        base = prev2 + cumbt
        r0 = jnp.sum(base * oh0, axis=1).astype(jnp.int32)
        r1 = jnp.sum(base * oh1, axis=1).astype(jnp.int32)
        bfb = jnp.broadcast_to(bf, (TB, NE))
        bf0 = jnp.sum(bfb * oh0, axis=1).astype(jnp.int32)
        bf1 = jnp.sum(bfb * oh1, axis=1).astype(jnp.int32)
        p0_ref[...] = ((bf0 + (r0 >> 9)) << 9) + (r0 & (BLK - 1))
        p1_ref[...] = ((bf1 + (r1 >> 9)) << 9) + (r1 & (BLK - 1))
        w0_ref[...] = w0s[g, :]
        w1_ref[...] = w1s[g, :]
        cnt2[...] = prev2 + jnp.sum(ohs, axis=0, keepdims=True)
        bidx = lax.broadcasted_iota(jnp.int32, (NG, NE), 0)
        bfg = jnp.broadcast_to(bf, (NG, NE)).astype(jnp.int32)
        be_ref[...] = jnp.clip(
            jnp.sum((bidx >= bfg).astype(jnp.int32), axis=1) - 1, 0, NE - 1)


def _route(xf, Wg, bg):
    return pl.pallas_call(
        _route_body,
        grid=(2, NTB),
        in_specs=[
            pl.BlockSpec((TB, DM), lambda p, g: (g * (1 - p), 0)),
            pl.BlockSpec((DM, NE), lambda p, g: (0, 0)),
            pl.BlockSpec((1, NE), lambda p, g: (0, 0)),
        ],
        out_specs=[
            pl.BlockSpec((TB,), lambda p, g: (g,)),
            pl.BlockSpec((TB,), lambda p, g: (g,)),
            pl.BlockSpec((TB,), lambda p, g: (g,)),
            pl.BlockSpec((TB,), lambda p, g: (g,)),
            pl.BlockSpec((NG,), lambda p, g: (0,)),
        ],
        out_shape=[
            jax.ShapeDtypeStruct((T,), jnp.float32),
            jax.ShapeDtypeStruct((T,), jnp.float32),
            jax.ShapeDtypeStruct((T,), jnp.int32),
            jax.ShapeDtypeStruct((T,), jnp.int32),
            jax.ShapeDtypeStruct((NG,), jnp.int32),
        ],
        scratch_shapes=[
            pltpu.VMEM((NTB, TB), jnp.int32),
            pltpu.VMEM((NTB, TB), jnp.int32),
            pltpu.VMEM((NTB, TB), jnp.float32),
            pltpu.VMEM((NTB, TB), jnp.float32),
            pltpu.VMEM((1, NE), jnp.float32),
            pltpu.VMEM((1, NE), jnp.float32),
        ],
    )(xf, Wg, bg.reshape(1, NE))


def _ffn_body(be_ref, xg_ref, wg_ref, bg_ref, wu_ref, bu_ref, wd_ref, bd_ref,
              ws_ref, og_ref):
    xb = xg_ref[...].astype(jnp.bfloat16)
    g = jnp.dot(xb, wg_ref[0], preferred_element_type=jnp.float32) + bg_ref[0]
    u = jnp.dot(xb, wu_ref[0], preferred_element_type=jnp.float32) + bu_ref[0]
    t = g * u
    h = (t * jax.nn.sigmoid(t)).astype(jnp.bfloat16)
    o = jnp.dot(h, wd_ref[0], preferred_element_type=jnp.float32) + bd_ref[0]
    og_ref[...] = o * ws_ref[...]


def _ffn(xg, W_gate, b_gate, W_up, b_up, W_down, b_down, w_slot, block_expert):
    grid_spec = pltpu.PrefetchScalarGridSpec(
        num_scalar_prefetch=1,
        grid=(NG,),
        in_specs=[
            pl.BlockSpec((BLK, DM), lambda g, be: (g, 0)),
            pl.BlockSpec((1, DM, DF), lambda g, be: (be[g], 0, 0)),
            pl.BlockSpec((1, 1, DF), lambda g, be: (be[g], 0, 0)),
            pl.BlockSpec((1, DM, DF), lambda g, be: (be[g], 0, 0)),
            pl.BlockSpec((1, 1, DF), lambda g, be: (be[g], 0, 0)),
            pl.BlockSpec((1, DF, DM), lambda g, be: (be[g], 0, 0)),
            pl.BlockSpec((1, 1, DM), lambda g, be: (be[g], 0, 0)),
            pl.BlockSpec((BLK, 1), lambda g, be: (g, 0)),
        ],
        out_specs=pl.BlockSpec((BLK, DM), lambda g, be: (g, 0)),
    )
    return pl.pallas_call(
        _ffn_body,
        grid_spec=grid_spec,
        out_shape=jax.ShapeDtypeStruct((S, DM), jnp.float32),
    )(block_expert, xg, W_gate, b_gate.reshape(NE, 1, DF), W_up,
      b_up.reshape(NE, 1, DF), W_down, b_down.reshape(NE, 1, DM),
      w_slot.reshape(S, 1))


NW = 32
TPW = T // NW
DCH = 64
CCH = 32

_MESH = plsc.VectorSubcoreMesh(core_axis_name="c", subcore_axis_name="s")


@functools.partial(
    pl.kernel, mesh=_MESH,
    out_type=[jax.ShapeDtypeStruct((S, DM), jnp.float32),
              jax.ShapeDtypeStruct((S,), jnp.float32)],
    scratch_types=[
        pltpu.VMEM((DCH, DM), jnp.float32),
        pltpu.VMEM((DCH,), jnp.int32),
        pltpu.VMEM((DCH,), jnp.int32),
        pltpu.VMEM((DCH,), jnp.float32),
        pltpu.VMEM((DCH,), jnp.float32),
        pltpu.SemaphoreType.DMA,
        pltpu.SemaphoreType.DMA,
        pltpu.SemaphoreType.DMA,
        pltpu.SemaphoreType.DMA,
    ],
)
def _sc_dispatch(x_hbm, p0_hbm, p1_hbm, w0_hbm, w1_hbm, xg_hbm, ws_hbm,
                 xbuf, p0v, p1v, w0v, w1v, s1, s2, s3, s4):
    wid = lax.axis_index("s") * 2 + lax.axis_index("c")
    base = wid * TPW

    def body(i, carry):
        off = base + i * DCH
        pltpu.sync_copy(x_hbm.at[pl.ds(off, DCH)], xbuf)
        pltpu.sync_copy(p0_hbm.at[pl.ds(off, DCH)], p0v)
        pltpu.sync_copy(p1_hbm.at[pl.ds(off, DCH)], p1v)
        pltpu.sync_copy(w0_hbm.at[pl.ds(off, DCH)], w0v)
        pltpu.sync_copy(w1_hbm.at[pl.ds(off, DCH)], w1v)
        c1 = pltpu.async_copy(xbuf, xg_hbm.at[p0v], s1)
        c2 = pltpu.async_copy(xbuf, xg_hbm.at[p1v], s2)
        c3 = pltpu.async_copy(w0v, ws_hbm.at[p0v], s3)
        c4 = pltpu.async_copy(w1v, ws_hbm.at[p1v], s4)
        c1.wait()
        c2.wait()
        c3.wait()
        c4.wait()
        return carry

    lax.fori_loop(0, TPW // DCH, body, 0)


@functools.partial(
    pl.kernel, mesh=_MESH,
    out_type=jax.ShapeDtypeStruct((T, DM), jnp.float32),
    scratch_types=[
        pltpu.VMEM((CCH, DM), jnp.float32),
        pltpu.VMEM((CCH, DM), jnp.float32),
        pltpu.VMEM((CCH,), jnp.int32),
        pltpu.VMEM((CCH,), jnp.int32),
        pltpu.SemaphoreType.DMA,
        pltpu.SemaphoreType.DMA,
    ],
)
def _sc_combine(og_hbm, p0_hbm, p1_hbm, y_hbm, bufa, bufb, p0v, p1v, sa, sb):
    wid = lax.axis_index("s") * 2 + lax.axis_index("c")
    base = wid * TPW

    def body(i, carry):
        off = base + i * CCH
        pltpu.sync_copy(p0_hbm.at[pl.ds(off, CCH)], p0v)
        pltpu.sync_copy(p1_hbm.at[pl.ds(off, CCH)], p1v)
        ca = pltpu.async_copy(og_hbm.at[p0v], bufa, sa)
        cb = pltpu.async_copy(og_hbm.at[p1v], bufb, sb)
        ca.wait()
        cb.wait()

        def add_row(j, c2):
            for k in range(DM // 16):
                sl = pl.ds(k * 16, 16)
                bufa[j, sl] = bufa[j, sl] + bufb[j, sl]
            return c2

        lax.fori_loop(0, CCH, add_row, 0)
        pltpu.sync_copy(bufa, y_hbm.at[pl.ds(off, CCH)])
        return carry

    lax.fori_loop(0, TPW // CCH, body, 0)


def kernel(x, Wg, bg, W_gate, b_gate, W_up, b_up, W_down, b_down):
    bs, slen, dim = x.shape
    xf = x.reshape(-1, dim)
    w0, w1, pos0, pos1, block_expert = _route(xf, Wg, bg)
    xg, w_slot = _sc_dispatch(xf, pos0, pos1, w0, w1)
    og = _ffn(xg, W_gate.astype(jnp.bfloat16), b_gate,
              W_up.astype(jnp.bfloat16), b_up,
              W_down.astype(jnp.bfloat16), b_down, w_slot, block_expert)
    y = _sc_combine(og, pos0, pos1)
    return y.reshape(bs, slen, dim)
    tok = jnp.arange(T, dtype=jnp.int32)
    tok_slot = (jnp.zeros((S,), jnp.int32).at[pos0].set(tok)
                .at[pos1].set(tok))
    w_slot = (jnp.zeros((S,), jnp.float32).at[pos0].set(w0)
              .at[pos1].set(w1))
    xbf = xf.astype(jnp.bfloat16)
    xg = xbf[tok_slot]
    og = _ffn(xg, W_gate.astype(jnp.bfloat16), b_gate,
              W_up.astype(jnp.bfloat16), b_up,
              W_down.astype(jnp.bfloat16), b_down, w_slot, block_expert)
    y = og[pos0] + og[pos1]
    return y.reshape(bs, slen, dim)
